# baseline (speedup 1.0000x reference)
import jax
import jax.numpy as jnp
from jax.experimental import pallas as pl

TOP_K = 8
N_EXPERTS = 72
N_LANES = 128
BT = 2048


def _router_block(hs_ref, ow_ref, nw_ref, ob_ref, nb_ref, out_ref):
    x = hs_ref[0]
    w = jnp.concatenate([ow_ref[...], nw_ref[...]], axis=0)
    bias = jnp.concatenate([ob_ref[...], nb_ref[...]], axis=1)
    logits = jax.lax.dot_general(
        x, w, (((1,), (1,)), ((), ())),
        preferred_element_type=jnp.float32) + bias

    lane_desc = (jnp.float32(127)
                 - jax.lax.broadcasted_iota(jnp.int32, (BT, N_EXPERTS), 1)
                 .astype(jnp.float32))
    neg = jnp.float32(-jnp.inf)
    cur = logits
    vals = []
    encs = []
    for _ in range(TOP_K):
        m = jnp.max(cur, axis=1, keepdims=True)
        enc = jnp.max(jnp.where(cur == m, lane_desc, neg), axis=1,
                      keepdims=True)
        vals.append(m)
        encs.append(enc)
        cur = jnp.where(lane_desc == enc, neg, cur)
    v = jnp.concatenate(vals, axis=1)
    e = jnp.concatenate(encs, axis=1)
    i_f = jnp.float32(127) - e
    sw = jax.nn.sigmoid(v)
    sw = sw / (jnp.sum(sw, axis=1, keepdims=True) + 1e-8)
    pad = jnp.zeros((BT, N_LANES - N_EXPERTS - 2 * TOP_K), jnp.float32)
    out_ref[0] = jnp.concatenate([logits, sw, i_f, pad], axis=1)


def kernel(hidden_states, orig_weight, orig_bias, new_weight, new_bias):
    b, s, d = hidden_states.shape
    ob = orig_bias.reshape(1, -1)
    nb = new_bias.reshape(1, -1)

    packed = pl.pallas_call(
        _router_block,
        grid=(b, s // BT),
        in_specs=[
            pl.BlockSpec((1, BT, d), lambda i, j: (i, j, 0)),
            pl.BlockSpec(orig_weight.shape, lambda i, j: (0, 0)),
            pl.BlockSpec(new_weight.shape, lambda i, j: (0, 0)),
            pl.BlockSpec(ob.shape, lambda i, j: (0, 0)),
            pl.BlockSpec(nb.shape, lambda i, j: (0, 0)),
        ],
        out_specs=pl.BlockSpec((1, BT, N_LANES), lambda i, j: (i, j, 0)),
        out_shape=jax.ShapeDtypeStruct((b, s, N_LANES), jnp.float32),
    )(hidden_states, orig_weight, new_weight, ob, nb)

    logits = packed[..., :N_EXPERTS]
    tw = packed[..., N_EXPERTS:N_EXPERTS + TOP_K]
    ti = packed[..., N_EXPERTS + TOP_K:N_EXPERTS + 2 * TOP_K].astype(jnp.int32)
    return (tw, ti, logits)


# speedup vs baseline: 1.9781x; 1.9781x over previous
import jax
import jax.numpy as jnp
from jax.experimental import pallas as pl

TOP_K = 8
N_EXPERTS = 72
BT = 2048


def _router_block(hs_ref, ow_ref, nw_ref, ob_ref, nb_ref,
                  lt_ref, tw_ref, ti_ref):
    x = hs_ref[0]
    w = jnp.concatenate([ow_ref[...], nw_ref[...]], axis=0)
    bias = jnp.concatenate([ob_ref[...], nb_ref[...]], axis=0)
    logits = jax.lax.dot_general(
        w, x, (((1,), (1,)), ((), ())),
        preferred_element_type=jnp.float32) + bias
    lt_ref[0] = logits

    row_desc = (jnp.float32(127)
                - jax.lax.broadcasted_iota(jnp.int32, (N_EXPERTS, BT), 0)
                .astype(jnp.float32))
    neg = jnp.float32(-jnp.inf)
    cur = logits
    vals = []
    encs = []
    for _ in range(TOP_K):
        m = jnp.max(cur, axis=0, keepdims=True)
        enc = jnp.max(jnp.where(cur == m, row_desc, neg), axis=0,
                      keepdims=True)
        vals.append(m)
        encs.append(enc)
        cur = jnp.where(row_desc == enc, neg, cur)
    v = jnp.concatenate(vals, axis=0)
    e = jnp.concatenate(encs, axis=0)
    sw = jax.nn.sigmoid(v)
    sw = sw / (jnp.sum(sw, axis=0, keepdims=True) + 1e-8)
    tw_ref[0] = sw
    ti_ref[0] = (jnp.float32(127) - e).astype(jnp.int32)


def kernel(hidden_states, orig_weight, orig_bias, new_weight, new_bias):
    b, s, d = hidden_states.shape
    ob = orig_bias.reshape(-1, 1)
    nb = new_bias.reshape(-1, 1)

    lt, tw, ti = pl.pallas_call(
        _router_block,
        grid=(b, s // BT),
        in_specs=[
            pl.BlockSpec((1, BT, d), lambda i, j: (i, j, 0)),
            pl.BlockSpec(orig_weight.shape, lambda i, j: (0, 0)),
            pl.BlockSpec(new_weight.shape, lambda i, j: (0, 0)),
            pl.BlockSpec(ob.shape, lambda i, j: (0, 0)),
            pl.BlockSpec(nb.shape, lambda i, j: (0, 0)),
        ],
        out_specs=[
            pl.BlockSpec((1, N_EXPERTS, BT), lambda i, j: (i, 0, j)),
            pl.BlockSpec((1, TOP_K, BT), lambda i, j: (i, 0, j)),
            pl.BlockSpec((1, TOP_K, BT), lambda i, j: (i, 0, j)),
        ],
        out_shape=[
            jax.ShapeDtypeStruct((b, N_EXPERTS, s), jnp.float32),
            jax.ShapeDtypeStruct((b, TOP_K, s), jnp.float32),
            jax.ShapeDtypeStruct((b, TOP_K, s), jnp.int32),
        ],
    )(hidden_states, orig_weight, new_weight, ob, nb)

    return (jnp.swapaxes(tw, 1, 2),
            jnp.swapaxes(ti, 1, 2),
            jnp.swapaxes(lt, 1, 2))
